# Initial kernel scaffold; baseline (speedup 1.0000x reference)
import jax
import jax.numpy as jnp
from jax.experimental import pallas as pl


def kernel(ids, table):
    raise NotImplementedError("write your pallas kernel here")


# speedup vs baseline: 6.6561x; 6.6561x over previous
import functools

import jax
import jax.numpy as jnp
from jax import lax
from jax.experimental import pallas as pl
from jax.experimental.pallas import tpu as pltpu
from jax.experimental.pallas import tpu_sc as plsc

B = 4096
H = 50
D = 64
L = 16
NC = 2
NS = 16
NW = NC * NS
GROUP = 2
IDX_PER_G = GROUP * H
RPW = B // NW
NG = RPW // GROUP
NGTOT = B // GROUP


def _tree_sum(vals):
    while len(vals) > 1:
        nxt = [vals[i] + vals[i + 1] for i in range(0, len(vals) - 1, 2)]
        if len(vals) % 2:
            nxt.append(vals[-1])
        vals = nxt
    return vals[0]


def _body(ids_hbm, table_hbm, out_hbm, idx_v, rows_v, out_v, sem):
    wid = lax.axis_index("s") * NC + lax.axis_index("c")
    gbase = wid * NG

    pltpu.sync_copy(ids_hbm.at[pl.ds(gbase, NG)], idx_v)

    def group_body(g, _):
        copy = pltpu.make_async_copy(table_hbm.at[idx_v.at[g]], rows_v, sem)
        copy.start()
        copy.wait()
        for r in range(GROUP):
            for d in range(D // L):
                vals = [rows_v[r * H + h, pl.ds(d * L, L)] for h in range(H)]
                out_v[g * GROUP + r, pl.ds(d * L, L)] = _tree_sum(vals)
        return 0

    lax.fori_loop(0, NG, group_body, 0)

    pltpu.sync_copy(out_v, out_hbm.at[pl.ds(wid * RPW, RPW)])


_embed_pool = functools.partial(
    pl.kernel,
    out_type=jax.ShapeDtypeStruct((B, D), jnp.float32),
    mesh=plsc.VectorSubcoreMesh(core_axis_name="c", subcore_axis_name="s"),
    scratch_types=[
        pltpu.VMEM((NG, IDX_PER_G), jnp.int32),
        pltpu.VMEM((IDX_PER_G, D), jnp.float32),
        pltpu.VMEM((RPW, D), jnp.float32),
        pltpu.SemaphoreType.DMA,
    ],
    compiler_params=pltpu.CompilerParams(use_tc_tiling_on_sc=False),
)(_body)


def kernel(ids, table):
    ids2 = ids.astype(jnp.int32).reshape(NGTOT, IDX_PER_G)
    return _embed_pool(ids2, table)


# speedup vs baseline: 10.3365x; 1.5529x over previous
import functools

import jax
import jax.numpy as jnp
from jax import lax
from jax.experimental import pallas as pl
from jax.experimental.pallas import tpu as pltpu
from jax.experimental.pallas import tpu_sc as plsc

B = 4096
H = 50
D = 64
L = 16
NC = 2
NS = 16
NW = NC * NS
RPW = B // NW


def _body(idsT_hbm, table_hbm, out_hbm, idxT_v, acc_v, sem):
    wid = lax.axis_index("s") * NC + lax.axis_index("c")
    base = wid * RPW

    pltpu.sync_copy(idsT_hbm.at[:, pl.ds(base, RPW)], idxT_v)

    zero = jnp.zeros((L,), jnp.float32)

    def zbody(i, _):
        for d in range(D // L):
            acc_v[i, pl.ds(d * L, L)] = zero
        return 0

    lax.fori_loop(0, RPW, zbody, 0)

    copies = [
        pltpu.async_copy(table_hbm.at[idxT_v.at[h]], acc_v, sem, add=True)
        for h in range(H)
    ]
    for c in copies:
        c.wait()

    pltpu.sync_copy(acc_v, out_hbm.at[pl.ds(base, RPW)])


_embed_pool = functools.partial(
    pl.kernel,
    out_type=jax.ShapeDtypeStruct((B, D), jnp.float32),
    mesh=plsc.VectorSubcoreMesh(core_axis_name="c", subcore_axis_name="s"),
    scratch_types=[
        pltpu.VMEM((H, RPW), jnp.int32),
        pltpu.VMEM((RPW, D), jnp.float32),
        pltpu.SemaphoreType.DMA,
    ],
    compiler_params=pltpu.CompilerParams(use_tc_tiling_on_sc=False),
)(_body)


def kernel(ids, table):
    idsT = ids.astype(jnp.int32).T
    return _embed_pool(idsT, table)
